# Initial kernel scaffold; baseline (speedup 1.0000x reference)
import jax
import jax.numpy as jnp
from jax.experimental import pallas as pl


def kernel(input_tensor, edge_index, node_degree_matrix, weight, bias):
    raise NotImplementedError("write your pallas kernel here")


# speedup vs baseline: 3.0925x; 3.0925x over previous
import jax
import jax.numpy as jnp
from jax import lax
from jax.experimental import pallas as pl
from jax.experimental.pallas import tpu as pltpu
from jax.experimental.pallas import tpu_sc as plsc

N_NODES = 10000
N_EDGES = 320000
D = 128

NC = 2
NS = 16
NW = NC * NS

CHUNK = 128
ACC_ROWS = 10240
ROWS_PER_TILE = ACC_ROWS // NS
INIT_STEPS = ROWS_PER_TILE // CHUNK
E_PER_TILE = 10240
N_CHUNKS = E_PER_TILE // CHUNK
E_PAD = NW * E_PER_TILE


def _sc_body(x_hbm, src_hbm, dst_hbm, z_hbm, out_hbm,
             acc, sidx, didx, rows, tmp, sem):
  cid = lax.axis_index("c")
  sid = lax.axis_index("s")
  wid = cid * NS + sid

  row0 = sid * ROWS_PER_TILE

  @pl.when(cid == 0)
  def _():
    def init(i, c):
      pltpu.sync_copy(x_hbm.at[pl.ds(row0 + i * CHUNK, CHUNK)], tmp)
      pltpu.sync_copy(tmp, acc.at[pl.ds(row0 + i * CHUNK, CHUNK)])
      return c
    lax.fori_loop(0, INIT_STEPS, init, 0)

  @pl.when(cid == 1)
  def _():
    pltpu.sync_copy(z_hbm, tmp)
    def init(i, c):
      pltpu.sync_copy(tmp, acc.at[pl.ds(row0 + i * CHUNK, CHUNK)])
      return c
    lax.fori_loop(0, INIT_STEPS, init, 0)

  plsc.subcore_barrier()

  base = wid * E_PER_TILE

  def chunk(j, c):
    e0 = base + j * CHUNK
    pltpu.sync_copy(src_hbm.at[pl.ds(e0, CHUNK)], sidx)
    pltpu.sync_copy(dst_hbm.at[pl.ds(e0, CHUNK)], didx)
    pltpu.async_copy(x_hbm.at[sidx], rows, sem).wait()
    pltpu.sync_copy(rows, acc.at[didx], add=True)
    return c

  lax.fori_loop(0, N_CHUNKS, chunk, 0)

  plsc.subcore_barrier()

  def out(i, c):
    pltpu.sync_copy(acc.at[pl.ds(row0 + i * CHUNK, CHUNK)], tmp)
    pltpu.sync_copy(tmp, out_hbm.at[cid, pl.ds(row0 + i * CHUNK, CHUNK)])
    return c
  lax.fori_loop(0, INIT_STEPS, out, 0)


_sc_agg = pl.kernel(
    _sc_body,
    out_type=jax.ShapeDtypeStruct((NC, ACC_ROWS, D), jnp.float32),
    mesh=plsc.VectorSubcoreMesh(
        core_axis_name="c", subcore_axis_name="s",
        num_cores=NC, num_subcores=NS),
    scratch_types=[
        pltpu.VMEM_SHARED((ACC_ROWS, D), jnp.float32),
        pltpu.VMEM((CHUNK,), jnp.int32),
        pltpu.VMEM((CHUNK,), jnp.int32),
        pltpu.VMEM((CHUNK, D), jnp.float32),
        pltpu.VMEM((CHUNK, D), jnp.float32),
        pltpu.SemaphoreType.DMA,
    ],
)


BR = 1000


def _tc_body(p_ref, w_ref, b_ref, deg_ref, o_ref):
  pool = p_ref[0] + p_ref[1]
  y = jnp.dot(pool, w_ref[...], preferred_element_type=jnp.float32)
  y = (y + b_ref[...]) / deg_ref[...]
  o_ref[...] = jnp.maximum(y, 0.0)


_tc_fin = pl.pallas_call(
    _tc_body,
    grid=(N_NODES // BR,),
    in_specs=[
        pl.BlockSpec((NC, BR, D), lambda i: (0, i, 0)),
        pl.BlockSpec((D, D), lambda i: (0, 0)),
        pl.BlockSpec((1, D), lambda i: (0, 0)),
        pl.BlockSpec((BR, 1), lambda i: (i, 0)),
    ],
    out_specs=pl.BlockSpec((BR, D), lambda i: (i, 0)),
    out_shape=jax.ShapeDtypeStruct((N_NODES, D), jnp.float32),
)


@jax.jit
def kernel(input_tensor, edge_index, node_degree_matrix, weight, bias):
  src = edge_index[0].astype(jnp.int32)
  dst = edge_index[1].astype(jnp.int32)
  npad = E_PAD - N_EDGES
  src = jnp.concatenate([src, jnp.zeros((npad,), jnp.int32)])
  dst = jnp.concatenate([dst, jnp.full((npad,), N_NODES, jnp.int32)])
  x_pad = jnp.concatenate(
      [input_tensor, jnp.zeros((ACC_ROWS - N_NODES, D), jnp.float32)])
  zeros = jnp.zeros((CHUNK, D), jnp.float32)
  partials = _sc_agg(x_pad, src, dst, zeros)
  return _tc_fin(partials, weight, bias.reshape(1, D), node_degree_matrix)


# speedup vs baseline: 3.4148x; 1.1042x over previous
import jax
import jax.numpy as jnp
from jax import lax
from jax.experimental import pallas as pl
from jax.experimental.pallas import tpu as pltpu
from jax.experimental.pallas import tpu_sc as plsc

N_NODES = 10000
N_EDGES = 320000
D = 128

NC = 2
NS = 16
NW = NC * NS

CHUNK = 128
E_PER_TILE = 10240
N_CHUNKS = E_PER_TILE // CHUNK
E_PAD = NW * E_PER_TILE

ACC_ROWS = 10112
ROWS_PER_TILE = ACC_ROWS // NS


def _sc_body(x_hbm, src_hbm, dst_hbm, z_hbm, out_hbm,
             acc, s0, s1, d0, d1, r0, r1, si0, si1, sr0, sr1):
  cid = lax.axis_index("c")
  sid = lax.axis_index("s")
  wid = cid * NS + sid
  sidx = (s0, s1)
  didx = (d0, d1)
  rows = (r0, r1)
  isem = (si0, si1)
  rsem = (sr0, sr1)

  base = wid * E_PER_TILE
  row0 = sid * ROWS_PER_TILE

  def fire_idx(jj, p):
    pltpu.async_copy(src_hbm.at[pl.ds(base + jj * CHUNK, CHUNK)], sidx[p], isem[p])
    pltpu.async_copy(dst_hbm.at[pl.ds(base + jj * CHUNK, CHUNK)], didx[p], isem[p])

  def wait_idx(jj, p):
    pltpu.make_async_copy(src_hbm.at[pl.ds(base + jj * CHUNK, CHUNK)], sidx[p], isem[p]).wait()
    pltpu.make_async_copy(dst_hbm.at[pl.ds(base + jj * CHUNK, CHUNK)], didx[p], isem[p]).wait()

  def fire_gather(p):
    pltpu.async_copy(x_hbm.at[sidx[p]], rows[p], rsem[p])

  def wait_gather(p):
    pltpu.make_async_copy(x_hbm.at[sidx[p]], rows[p], rsem[p]).wait()

  @pl.when(cid == 0)
  def _():
    def init(i, c):
      pltpu.sync_copy(x_hbm.at[pl.ds(row0 + i * CHUNK, CHUNK)], r0)
      pltpu.sync_copy(r0, acc.at[pl.ds(row0 + i * CHUNK, CHUNK)])
      return c
    lax.fori_loop(0, 4, init, 0)
    pltpu.sync_copy(x_hbm.at[pl.ds(row0 + 512, 120)], r0.at[pl.ds(0, 120)])
    pltpu.sync_copy(r0.at[pl.ds(0, 120)], acc.at[pl.ds(row0 + 512, 120)])

  @pl.when(cid == 1)
  def _():
    pltpu.sync_copy(z_hbm, r0)
    def init(i, c):
      pltpu.sync_copy(r0, acc.at[pl.ds(row0 + i * CHUNK, CHUNK)])
      return c
    lax.fori_loop(0, 4, init, 0)
    pltpu.sync_copy(r0.at[pl.ds(0, 120)], acc.at[pl.ds(row0 + 512, 120)])

  plsc.subcore_barrier()

  fire_idx(0, 0)
  wait_idx(0, 0)
  fire_gather(0)
  fire_idx(1, 1)

  def group(g, c):
    for b in range(2):
      jj = 2 * g + b
      wait_gather(b)
      pltpu.sync_copy(rows[b], acc.at[didx[b]], add=True)
      fire_idx(jj + 2, b)
      wait_idx(jj + 1, 1 - b)
      fire_gather(1 - b)
    return c

  lax.fori_loop(0, N_CHUNKS // 2 - 1, group, 0)

  wait_gather(0)
  pltpu.sync_copy(rows[0], acc.at[didx[0]], add=True)
  wait_idx(N_CHUNKS - 1, 1)
  fire_gather(1)
  wait_gather(1)
  pltpu.sync_copy(rows[1], acc.at[didx[1]], add=True)

  plsc.subcore_barrier()

  def out(i, c):
    pltpu.sync_copy(acc.at[pl.ds(row0 + i * CHUNK, CHUNK)], r0)
    pltpu.sync_copy(
        r0, out_hbm.at[pl.ds(cid * ACC_ROWS + row0 + i * CHUNK, CHUNK)])
    return c
  lax.fori_loop(0, 4, out, 0)
  pltpu.sync_copy(acc.at[pl.ds(row0 + 512, 120)], r0.at[pl.ds(0, 120)])
  pltpu.sync_copy(r0.at[pl.ds(0, 120)],
                  out_hbm.at[pl.ds(cid * ACC_ROWS + row0 + 512, 120)])


_sc_agg = pl.kernel(
    _sc_body,
    out_type=jax.ShapeDtypeStruct((NC * ACC_ROWS, D), jnp.float32),
    mesh=plsc.VectorSubcoreMesh(
        core_axis_name="c", subcore_axis_name="s",
        num_cores=NC, num_subcores=NS),
    scratch_types=[
        pltpu.VMEM_SHARED((ACC_ROWS, D), jnp.float32),
        pltpu.VMEM((CHUNK,), jnp.int32),
        pltpu.VMEM((CHUNK,), jnp.int32),
        pltpu.VMEM((CHUNK,), jnp.int32),
        pltpu.VMEM((CHUNK,), jnp.int32),
        pltpu.VMEM((CHUNK, D), jnp.float32),
        pltpu.VMEM((CHUNK, D), jnp.float32),
        pltpu.SemaphoreType.DMA,
        pltpu.SemaphoreType.DMA,
        pltpu.SemaphoreType.DMA,
        pltpu.SemaphoreType.DMA,
    ],
)


BR = 1000


def _tc_body(p_ref, w_ref, b_ref, deg_ref, o_ref):
  pool = p_ref[0] + p_ref[1]
  y = jnp.dot(pool, w_ref[...], preferred_element_type=jnp.float32)
  y = (y + b_ref[...]) / deg_ref[...]
  o_ref[...] = jnp.maximum(y, 0.0)


_tc_fin = pl.pallas_call(
    _tc_body,
    grid=(N_NODES // BR,),
    in_specs=[
        pl.BlockSpec((NC, BR, D), lambda i: (0, i, 0)),
        pl.BlockSpec((D, D), lambda i: (0, 0)),
        pl.BlockSpec((1, D), lambda i: (0, 0)),
        pl.BlockSpec((BR, 1), lambda i: (i, 0)),
    ],
    out_specs=pl.BlockSpec((BR, D), lambda i: (i, 0)),
    out_shape=jax.ShapeDtypeStruct((N_NODES, D), jnp.float32),
)


@jax.jit
def kernel(input_tensor, edge_index, node_degree_matrix, weight, bias):
  src = edge_index[0].astype(jnp.int32)
  dst = edge_index[1].astype(jnp.int32)
  npad = E_PAD - N_EDGES
  src = jnp.concatenate([src, jnp.zeros((npad,), jnp.int32)])
  dst = jnp.concatenate([dst, jnp.full((npad,), N_NODES, jnp.int32)])
  x_pad = jnp.concatenate(
      [input_tensor, jnp.zeros((ACC_ROWS - N_NODES, D), jnp.float32)])
  zeros = jnp.zeros((CHUNK, D), jnp.float32)
  partials = _sc_agg(x_pad, src, dst, zeros).reshape(NC, ACC_ROWS, D)
  return _tc_fin(partials, weight, bias.reshape(1, D), node_degree_matrix)


# speedup vs baseline: 8.4950x; 2.4877x over previous
import jax
import jax.numpy as jnp
from jax import lax
from jax.experimental import pallas as pl
from jax.experimental.pallas import tpu as pltpu
from jax.experimental.pallas import tpu_sc as plsc

N_NODES = 10000
N_EDGES = 320000
D = 128

NC = 2
NS = 16
NW = NC * NS

CHUNK = 128
E_PER_TILE = 10240
N_CHUNKS = E_PER_TILE // CHUNK
E_PAD = NW * E_PER_TILE

ACC_ROWS = 10112
ROWS_PER_TILE = ACC_ROWS // NS


def _sc_body(x_hbm, src_hbm, dst_hbm, z_hbm, out_hbm,
             acc, s0, s1, d0, d1, r0, r1, si0, si1, sr0, sr1):
  cid = lax.axis_index("c")
  sid = lax.axis_index("s")
  wid = cid * NS + sid
  sidx = (s0, s1)
  didx = (d0, d1)
  rows = (r0, r1)
  isem = (si0, si1)
  rsem = (sr0, sr1)

  base = wid * E_PER_TILE
  row0 = sid * ROWS_PER_TILE

  def fire_idx(jj, p):
    pltpu.async_copy(src_hbm.at[pl.ds(base + jj * CHUNK, CHUNK)], sidx[p], isem[p])
    pltpu.async_copy(dst_hbm.at[pl.ds(base + jj * CHUNK, CHUNK)], didx[p], isem[p])

  def wait_idx(jj, p):
    pltpu.make_async_copy(src_hbm.at[pl.ds(base + jj * CHUNK, CHUNK)], sidx[p], isem[p]).wait()
    pltpu.make_async_copy(dst_hbm.at[pl.ds(base + jj * CHUNK, CHUNK)], didx[p], isem[p]).wait()

  def fire_gather(p):
    pltpu.async_copy(x_hbm.at[sidx[p]], rows[p], rsem[p])

  def wait_gather(p):
    pltpu.make_async_copy(x_hbm.at[sidx[p]], rows[p], rsem[p]).wait()

  @pl.when(cid == 0)
  def _():
    def init(i, c):
      pltpu.sync_copy(x_hbm.at[pl.ds(row0 + i * CHUNK, CHUNK)], r0)
      pltpu.sync_copy(r0, acc.at[pl.ds(row0 + i * CHUNK, CHUNK)])
      return c
    lax.fori_loop(0, 4, init, 0)
    pltpu.sync_copy(x_hbm.at[pl.ds(row0 + 512, 120)], r0.at[pl.ds(0, 120)])
    pltpu.sync_copy(r0.at[pl.ds(0, 120)], acc.at[pl.ds(row0 + 512, 120)])

  @pl.when(cid == 1)
  def _():
    pltpu.sync_copy(z_hbm, r0)
    def init(i, c):
      pltpu.sync_copy(r0, acc.at[pl.ds(row0 + i * CHUNK, CHUNK)])
      return c
    lax.fori_loop(0, 4, init, 0)
    pltpu.sync_copy(r0.at[pl.ds(0, 120)], acc.at[pl.ds(row0 + 512, 120)])

  plsc.subcore_barrier()

  fire_idx(0, 0)
  wait_idx(0, 0)
  fire_gather(0)
  fire_idx(1, 1)

  def group(g, c):
    for b in range(2):
      jj = 2 * g + b
      wait_gather(b)
      pltpu.sync_copy(rows[b], acc.at[didx[b]], add=True)
      fire_idx(jj + 2, b)
      wait_idx(jj + 1, 1 - b)
      fire_gather(1 - b)
    return c

  lax.fori_loop(0, N_CHUNKS // 2 - 1, group, 0)

  wait_gather(0)
  pltpu.sync_copy(rows[0], acc.at[didx[0]], add=True)
  wait_idx(N_CHUNKS - 1, 1)
  fire_gather(1)
  wait_gather(1)
  pltpu.sync_copy(rows[1], acc.at[didx[1]], add=True)

  plsc.subcore_barrier()

  def out(i, c):
    pltpu.sync_copy(acc.at[pl.ds(row0 + i * CHUNK, CHUNK)], r0)
    pltpu.sync_copy(
        r0, out_hbm.at[pl.ds(cid * ACC_ROWS + row0 + i * CHUNK, CHUNK)])
    return c
  lax.fori_loop(0, 4, out, 0)
  pltpu.sync_copy(acc.at[pl.ds(row0 + 512, 120)], r0.at[pl.ds(0, 120)])
  pltpu.sync_copy(r0.at[pl.ds(0, 120)],
                  out_hbm.at[pl.ds(cid * ACC_ROWS + row0 + 512, 120)])


_sc_agg = pl.kernel(
    _sc_body,
    out_type=jax.ShapeDtypeStruct((NC * ACC_ROWS, D), jnp.float32),
    mesh=plsc.VectorSubcoreMesh(
        core_axis_name="c", subcore_axis_name="s",
        num_cores=NC, num_subcores=NS),
    scratch_types=[
        pltpu.VMEM_SHARED((ACC_ROWS, D), jnp.float32),
        pltpu.VMEM((CHUNK,), jnp.int32),
        pltpu.VMEM((CHUNK,), jnp.int32),
        pltpu.VMEM((CHUNK,), jnp.int32),
        pltpu.VMEM((CHUNK,), jnp.int32),
        pltpu.VMEM((CHUNK, D), jnp.float32),
        pltpu.VMEM((CHUNK, D), jnp.float32),
        pltpu.SemaphoreType.DMA,
        pltpu.SemaphoreType.DMA,
        pltpu.SemaphoreType.DMA,
        pltpu.SemaphoreType.DMA,
    ],
)


BR = 1000


def _tc_body(p_ref, w_ref, b_ref, deg_ref, o_ref):
  pool = p_ref[0] + p_ref[1]
  y = jnp.dot(pool, w_ref[...], preferred_element_type=jnp.float32)
  y = (y + b_ref[...]) / deg_ref[...]
  o_ref[...] = jnp.maximum(y, 0.0)


_tc_fin = pl.pallas_call(
    _tc_body,
    grid=(N_NODES // BR,),
    in_specs=[
        pl.BlockSpec((NC, BR, D), lambda i: (0, i, 0)),
        pl.BlockSpec((D, D), lambda i: (0, 0)),
        pl.BlockSpec((1, D), lambda i: (0, 0)),
        pl.BlockSpec((BR, 1), lambda i: (i, 0)),
    ],
    out_specs=pl.BlockSpec((BR, D), lambda i: (i, 0)),
    out_shape=jax.ShapeDtypeStruct((N_NODES, D), jnp.float32),
)


@jax.jit
def kernel(input_tensor, edge_index, node_degree_matrix, weight, bias):
  src = edge_index[0].astype(jnp.int32)
  dst = edge_index[1].astype(jnp.int32)
  npad = E_PAD - N_EDGES
  k = jnp.arange(npad, dtype=jnp.int32)
  src = jnp.concatenate([src, k % N_NODES])
  dst = jnp.concatenate([dst, N_NODES + (k % (ACC_ROWS - N_NODES))])
  x_pad = jnp.concatenate(
      [input_tensor, jnp.zeros((ACC_ROWS - N_NODES, D), jnp.float32)])
  zeros = jnp.zeros((CHUNK, D), jnp.float32)
  partials = _sc_agg(x_pad, src, dst, zeros).reshape(NC, ACC_ROWS, D)
  return _tc_fin(partials, weight, bias.reshape(1, D), node_degree_matrix)


# speedup vs baseline: 10.8338x; 1.2753x over previous
import jax
import jax.numpy as jnp
from jax import lax
from jax.experimental import pallas as pl
from jax.experimental.pallas import tpu as pltpu
from jax.experimental.pallas import tpu_sc as plsc

N_NODES = 10000
N_EDGES = 320000
D = 128

NC = 2
NS = 16
NW = NC * NS

CHUNK = 128
E_PER_TILE = 10240
N_CHUNKS = E_PER_TILE // CHUNK
E_PAD = NW * E_PER_TILE

ACC_ROWS = 10112
ROWS_PER_TILE = ACC_ROWS // NS


def _sc_body(x_hbm, src_hbm, dst_hbm, z_hbm, out_hbm,
             acc, s0, s1, s2, s3, d0, d1, d2, d3, r0, r1,
             si0, si1, si2, si3, sr0, sr1, ss0, ss1):
  cid = lax.axis_index("c")
  sid = lax.axis_index("s")
  wid = cid * NS + sid
  sidx = (s0, s1, s2, s3)
  didx = (d0, d1, d2, d3)
  rows = (r0, r1)
  isem = (si0, si1, si2, si3)
  rsem = (sr0, sr1)
  ssem = (ss0, ss1)

  base = wid * E_PER_TILE
  row0 = sid * ROWS_PER_TILE

  def fire_idx(jj, q):
    pltpu.async_copy(src_hbm.at[pl.ds(base + jj * CHUNK, CHUNK)], sidx[q], isem[q])
    pltpu.async_copy(dst_hbm.at[pl.ds(base + jj * CHUNK, CHUNK)], didx[q], isem[q])

  def wait_idx(jj, q):
    pltpu.make_async_copy(src_hbm.at[pl.ds(base + jj * CHUNK, CHUNK)], sidx[q], isem[q]).wait()
    pltpu.make_async_copy(dst_hbm.at[pl.ds(base + jj * CHUNK, CHUNK)], didx[q], isem[q]).wait()

  def fire_gather(p, q):
    pltpu.async_copy(x_hbm.at[sidx[q]], rows[p], rsem[p])

  def wait_gather(p, q):
    pltpu.make_async_copy(x_hbm.at[sidx[q]], rows[p], rsem[p]).wait()

  def fire_scatter(p, q):
    pltpu.async_copy(rows[p], acc.at[didx[q]], ssem[p], add=True)

  def wait_scatter(p, q):
    pltpu.make_async_copy(rows[p], acc.at[didx[q]], ssem[p]).wait()

  @pl.when(cid == 0)
  def _():
    def init(i, c):
      pltpu.sync_copy(x_hbm.at[pl.ds(row0 + i * CHUNK, CHUNK)], r0)
      pltpu.sync_copy(r0, acc.at[pl.ds(row0 + i * CHUNK, CHUNK)])
      return c
    lax.fori_loop(0, 4, init, 0)
    @pl.when(sid < NS - 1)
    def _():
      pltpu.sync_copy(x_hbm.at[pl.ds(row0 + 512, 120)], r0.at[pl.ds(0, 120)])
      pltpu.sync_copy(r0.at[pl.ds(0, 120)], acc.at[pl.ds(row0 + 512, 120)])
    @pl.when(sid == NS - 1)
    def _():
      pltpu.sync_copy(x_hbm.at[pl.ds(row0 + 512, 8)], r0.at[pl.ds(0, 8)])
      pltpu.sync_copy(r0.at[pl.ds(0, 8)], acc.at[pl.ds(row0 + 512, 8)])

  @pl.when(cid == 1)
  def _():
    pltpu.sync_copy(z_hbm, r0)
    def init(i, c):
      pltpu.sync_copy(r0, acc.at[pl.ds(row0 + i * CHUNK, CHUNK)])
      return c
    lax.fori_loop(0, 4, init, 0)
    @pl.when(sid < NS - 1)
    def _():
      pltpu.sync_copy(r0.at[pl.ds(0, 120)], acc.at[pl.ds(row0 + 512, 120)])
    @pl.when(sid == NS - 1)
    def _():
      pltpu.sync_copy(r0.at[pl.ds(0, 8)], acc.at[pl.ds(row0 + 512, 8)])

  plsc.subcore_barrier()

  fire_idx(0, 0)
  wait_idx(0, 0)
  fire_gather(0, 0)
  fire_idx(1, 1)
  fire_idx(2, 2)

  wait_gather(0, 0)
  fire_scatter(0, 0)
  fire_idx(3, 3)
  wait_idx(1, 1)
  fire_gather(1, 1)
  for j in (1, 2, 3):
    p, q = j % 2, j % 4
    wait_gather(p, q)
    wait_scatter(1 - p, (q + 3) % 4)
    fire_scatter(p, q)
    fire_idx(j + 3, (q + 3) % 4)
    wait_idx(j + 1, (q + 1) % 4)
    fire_gather(1 - p, (q + 1) % 4)

  def group(g, c):
    j0 = 4 * g
    for b in range(4):
      j = j0 + b
      p, q = b % 2, b
      wait_gather(p, q)
      wait_scatter(1 - p, (q + 3) % 4)
      fire_scatter(p, q)
      fire_idx(j + 3, (q + 3) % 4)
      wait_idx(j + 1, (q + 1) % 4)
      fire_gather(1 - p, (q + 1) % 4)
    return c

  lax.fori_loop(1, N_CHUNKS // 4 - 1, group, 0)

  for j in (N_CHUNKS - 4, N_CHUNKS - 3, N_CHUNKS - 2, N_CHUNKS - 1):
    p, q = j % 2, j % 4
    wait_gather(p, q)
    wait_scatter(1 - p, (q + 3) % 4)
    fire_scatter(p, q)
    if j + 3 < N_CHUNKS:
      fire_idx(j + 3, (q + 3) % 4)
    if j + 1 < N_CHUNKS:
      wait_idx(j + 1, (q + 1) % 4)
      fire_gather(1 - p, (q + 1) % 4)
  wait_scatter((N_CHUNKS - 1) % 2, (N_CHUNKS - 1) % 4)

  plsc.subcore_barrier()

  def out(i, c):
    pltpu.sync_copy(acc.at[pl.ds(row0 + i * CHUNK, CHUNK)], r0)
    pltpu.sync_copy(
        r0, out_hbm.at[pl.ds(cid * ACC_ROWS + row0 + i * CHUNK, CHUNK)])
    return c
  lax.fori_loop(0, 4, out, 0)
  pltpu.sync_copy(acc.at[pl.ds(row0 + 512, 120)], r0.at[pl.ds(0, 120)])
  pltpu.sync_copy(r0.at[pl.ds(0, 120)],
                  out_hbm.at[pl.ds(cid * ACC_ROWS + row0 + 512, 120)])


_sc_agg = pl.kernel(
    _sc_body,
    out_type=jax.ShapeDtypeStruct((NC * ACC_ROWS, D), jnp.float32),
    mesh=plsc.VectorSubcoreMesh(
        core_axis_name="c", subcore_axis_name="s",
        num_cores=NC, num_subcores=NS),
    scratch_types=[
        pltpu.VMEM_SHARED((ACC_ROWS, D), jnp.float32),
        pltpu.VMEM((CHUNK,), jnp.int32),
        pltpu.VMEM((CHUNK,), jnp.int32),
        pltpu.VMEM((CHUNK,), jnp.int32),
        pltpu.VMEM((CHUNK,), jnp.int32),
        pltpu.VMEM((CHUNK,), jnp.int32),
        pltpu.VMEM((CHUNK,), jnp.int32),
        pltpu.VMEM((CHUNK,), jnp.int32),
        pltpu.VMEM((CHUNK,), jnp.int32),
        pltpu.VMEM((CHUNK, D), jnp.float32),
        pltpu.VMEM((CHUNK, D), jnp.float32),
        pltpu.SemaphoreType.DMA,
        pltpu.SemaphoreType.DMA,
        pltpu.SemaphoreType.DMA,
        pltpu.SemaphoreType.DMA,
        pltpu.SemaphoreType.DMA,
        pltpu.SemaphoreType.DMA,
        pltpu.SemaphoreType.DMA,
        pltpu.SemaphoreType.DMA,
    ],
)


BR = 1000


def _tc_body(p_ref, w_ref, b_ref, deg_ref, o_ref):
  pool = p_ref[0] + p_ref[1]
  y = jnp.dot(pool, w_ref[...], preferred_element_type=jnp.float32)
  y = (y + b_ref[...]) / deg_ref[...]
  o_ref[...] = jnp.maximum(y, 0.0)


_tc_fin = pl.pallas_call(
    _tc_body,
    grid=(N_NODES // BR,),
    in_specs=[
        pl.BlockSpec((NC, BR, D), lambda i: (0, i, 0)),
        pl.BlockSpec((D, D), lambda i: (0, 0)),
        pl.BlockSpec((1, D), lambda i: (0, 0)),
        pl.BlockSpec((BR, 1), lambda i: (i, 0)),
    ],
    out_specs=pl.BlockSpec((BR, D), lambda i: (i, 0)),
    out_shape=jax.ShapeDtypeStruct((N_NODES, D), jnp.float32),
)


@jax.jit
def kernel(input_tensor, edge_index, node_degree_matrix, weight, bias):
  src = edge_index[0].astype(jnp.int32)
  dst = edge_index[1].astype(jnp.int32)
  npad = E_PAD - N_EDGES
  k = jnp.arange(npad, dtype=jnp.int32)
  src = jnp.concatenate([src, k % N_NODES])
  dst = jnp.concatenate([dst, N_NODES + (k % (ACC_ROWS - N_NODES))])
  zeros = jnp.zeros((CHUNK, D), jnp.float32)
  partials = _sc_agg(input_tensor, src, dst, zeros).reshape(NC, ACC_ROWS, D)
  return _tc_fin(partials, weight, bias.reshape(1, D), node_degree_matrix)
